# baseline (speedup 1.0000x reference)
import functools

import jax
import jax.numpy as jnp
from jax import lax
from jax.experimental import pallas as pl
from jax.experimental.pallas import tpu as pltpu
from jax.experimental.pallas import tpu_sc as plsc

N_NODES = 10000
D_OUT = 128
NC = 2
NS = 16
C = 80
NB = 4
NQ = 8
BM = 1000


def _matmul_body(x_ref, w_ref, o_ref):
    o_ref[...] = jnp.dot(x_ref[...], w_ref[...],
                         preferred_element_type=jnp.float32)


def _matmul(x, w):
    m, k = x.shape
    _, n = w.shape
    return pl.pallas_call(
        _matmul_body,
        grid=(m // BM,),
        in_specs=[
            pl.BlockSpec((BM, k), lambda i: (i, 0)),
            pl.BlockSpec((k, n), lambda i: (0, 0)),
        ],
        out_specs=pl.BlockSpec((BM, n), lambda i: (i, 0)),
        out_shape=jax.ShapeDtypeStruct((m, n), jnp.float32),
    )(x, w)


def _combine2_body(a_ref, b_ref, o_ref):
    o_ref[...] = a_ref[...] + b_ref[...]


def _combine2(parts):
    return pl.pallas_call(
        _combine2_body,
        grid=(N_NODES // BM,),
        in_specs=[
            pl.BlockSpec((BM, D_OUT), lambda i: (i, 0)),
            pl.BlockSpec((BM, D_OUT), lambda i: (i + N_NODES // BM, 0)),
        ],
        out_specs=pl.BlockSpec((BM, D_OUT), lambda i: (i, 0)),
        out_shape=jax.ShapeDtypeStruct((N_NODES, D_OUT), jnp.float32),
    )(parts, parts)


def _final_body(z_ref, a_ref, b_ref, bias_ref, o_ref):
    o_ref[...] = z_ref[...] + a_ref[...] + b_ref[...] + bias_ref[...]


def _final(z1, parts2, bias2d):
    return pl.pallas_call(
        _final_body,
        grid=(N_NODES // BM,),
        in_specs=[
            pl.BlockSpec((BM, D_OUT), lambda i: (i, 0)),
            pl.BlockSpec((BM, D_OUT), lambda i: (i, 0)),
            pl.BlockSpec((BM, D_OUT), lambda i: (i + N_NODES // BM, 0)),
            pl.BlockSpec((1, D_OUT), lambda i: (0, 0)),
        ],
        out_specs=pl.BlockSpec((BM, D_OUT), lambda i: (i, 0)),
        out_shape=jax.ShapeDtypeStruct((N_NODES, D_OUT), jnp.float32),
    )(z1, parts2, parts2, bias2d)


def _make_spmm(n_edges):
    ept = n_edges // (NC * NS)
    rpt = ept // C
    mesh = plsc.VectorSubcoreMesh(core_axis_name="c", subcore_axis_name="s")

    ZR = 40
    scratch = (
        [pltpu.VMEM((C, D_OUT), jnp.float32) for _ in range(NB)]
        + [pltpu.VMEM((C,), jnp.int32) for _ in range(NQ)]
        + [pltpu.VMEM((C,), jnp.int32) for _ in range(NQ)]
        + [pltpu.VMEM((C,), jnp.float32) for _ in range(NQ)]
        + [pltpu.VMEM((ZR, D_OUT), jnp.float32)]
        + [pltpu.VMEM_SHARED((N_NODES, D_OUT), jnp.float32)]
        + [pltpu.SemaphoreType.DMA] * (2 * NB + NQ + 1)
    )

    @functools.partial(
        pl.kernel,
        out_type=jax.ShapeDtypeStruct((NC * N_NODES, D_OUT), jnp.float32),
        mesh=mesh,
        scratch_types=scratch,
    )
    def spmm(table, dst_e, src_e, w_e, out, *scr):
        rows = scr[:NB]
        src_v = scr[NB:NB + NQ]
        dst_v = scr[NB + NQ:NB + 2 * NQ]
        w_v = scr[NB + 2 * NQ:NB + 3 * NQ]
        zbuf = scr[NB + 3 * NQ]
        acc = scr[NB + 3 * NQ + 1]
        sems = scr[NB + 3 * NQ + 2:]
        gsem = sems[:NB]
        ssem = sems[NB:2 * NB]
        isem = sems[2 * NB:2 * NB + NQ]
        zsem = sems[2 * NB + NQ]
        c = lax.axis_index("c")
        s = lax.axis_index("s")
        wid = c * NS + s
        base = wid * ept
        nrow_chunks = N_NODES // C
        nz = N_NODES // ZR

        def start_idx(i, q):
            off = base + i * C
            pltpu.async_copy(src_e.at[pl.ds(off, C)], src_v[q], isem[q])
            pltpu.async_copy(dst_e.at[pl.ds(off, C)], dst_v[q], isem[q])
            pltpu.async_copy(w_e.at[pl.ds(off, C)], w_v[q], isem[q])

        def wait_idx(i, q):
            off = base + i * C
            pltpu.make_async_copy(src_e.at[pl.ds(off, C)], src_v[q],
                                  isem[q]).wait()
            pltpu.make_async_copy(dst_e.at[pl.ds(off, C)], dst_v[q],
                                  isem[q]).wait()
            pltpu.make_async_copy(w_e.at[pl.ds(off, C)], w_v[q],
                                  isem[q]).wait()

        def start_gather(b, q):
            pltpu.async_copy(table.at[src_v[q]], rows[b], gsem[b])

        def wait_gather(b, q):
            pltpu.make_async_copy(table.at[src_v[q]], rows[b],
                                  gsem[b]).wait()

        def start_scatter(b, q):
            pltpu.async_copy(rows[b], acc.at[dst_v[q]], ssem[b], add=True)

        def wait_scatter(b, q):
            pltpu.make_async_copy(rows[b], acc.at[dst_v[q]],
                                  ssem[b]).wait()

        def scale(b, q):
            def group(g, inner):
                wvec = w_v[q][pl.ds(g * 16, 16)]
                for e16 in range(16):
                    e = g * 16 + e16
                    ws = wvec[e16]
                    for j in range(D_OUT // 16):
                        sl = pl.ds(j * 16, 16)
                        rows[b][e, sl] = rows[b][e, sl] * ws
                return inner
            lax.fori_loop(0, C // 16, group, 0)

        for i in range(4):
            start_idx(i, i)
        for i in range(2):
            wait_idx(i, i)
            start_gather(i, i)

        def zfill(e, carry):
            for j in range(D_OUT // 16):
                zbuf[e, pl.ds(j * 16, 16)] = jnp.zeros((16,), jnp.float32)
            return carry
        lax.fori_loop(0, ZR, zfill, 0)

        def zcopy(k, carry):
            idx = s + k * NS

            @pl.when(idx < nz)
            def _():
                pltpu.async_copy(zbuf, acc.at[pl.ds(idx * ZR, ZR)], zsem)
            return carry
        lax.fori_loop(0, pl.cdiv(nz, NS), zcopy, 0)

        def zdrain(k, carry):
            idx = s + k * NS

            @pl.when(idx < nz)
            def _():
                pltpu.make_async_copy(zbuf, acc.at[pl.ds(idx * ZR, ZR)],
                                      zsem).wait()
            return carry
        lax.fori_loop(0, pl.cdiv(nz, NS), zdrain, 0)
        plsc.subcore_barrier()

        def oct_body(k, carry):
            for u in range(8):
                ch = 8 * k + u
                b = u % NB
                wait_gather(b, u)
                scale(b, u)
                start_scatter(b, u)
                wm = (u + 2) % NB
                if u < 2:
                    @pl.when(k > 0)
                    def _(wm=wm, u=u):
                        wait_scatter(wm, (u - 2) % NQ)
                else:
                    wait_scatter(wm, (u - 2) % NQ)
                start_idx(ch + 4, (u + 4) % NQ)
                wait_idx(ch + 2, (u + 2) % NQ)
                start_gather((u + 2) % NB, (u + 2) % NQ)
            return carry
        lax.fori_loop(0, (rpt - 5) // 8, oct_body, 0)

        for ch in range(rpt - 5, rpt):
            u = ch % NQ
            b = u % NB
            wait_gather(b, u)
            scale(b, u)
            start_scatter(b, u)
            if ch + 2 < rpt:
                wait_scatter((u + 2) % NB, (u - 2) % NQ)
                if ch + 4 < rpt:
                    start_idx(ch + 4, (u + 4) % NQ)
                wait_idx(ch + 2, (u + 2) % NQ)
                start_gather((u + 2) % NB, (u + 2) % NQ)
        for ch in range(rpt - 4, rpt):
            u = ch % NQ
            wait_scatter(u % NB, u)
        plsc.subcore_barrier()

        def pubcopy(k, carry):
            idx = s + k * NS

            @pl.when(idx < nrow_chunks)
            def _():
                pltpu.async_copy(acc.at[pl.ds(idx * C, C)],
                                 out.at[pl.ds(c * N_NODES + idx * C, C)],
                                 gsem[0])
            return carry
        lax.fori_loop(0, pl.cdiv(nrow_chunks, NS), pubcopy, 0)

        def pubdrain(k, carry):
            idx = s + k * NS

            @pl.when(idx < nrow_chunks)
            def _():
                pltpu.make_async_copy(
                    acc.at[pl.ds(idx * C, C)],
                    out.at[pl.ds(c * N_NODES + idx * C, C)], gsem[0]).wait()
            return carry
        lax.fori_loop(0, pl.cdiv(nrow_chunks, NS), pubdrain, 0)

    return spmm


def kernel(x, edge_index, edge_weight, W, b):
    dst = edge_index[0]
    src = edge_index[1]
    n_edges = edge_weight.shape[0]

    y = _matmul(x, W)
    spmm = _make_spmm(n_edges)
    parts1 = spmm(y, dst, src, edge_weight)
    z1 = _combine2(parts1)
    parts2 = spmm(z1, dst, src, edge_weight)
    bias2d = jnp.reshape(b, (1, D_OUT))
    return _final(z1, parts2, bias2d)


# speedup vs baseline: 1.0189x; 1.0189x over previous
import functools

import jax
import jax.numpy as jnp
from jax import lax
from jax.experimental import pallas as pl
from jax.experimental.pallas import tpu as pltpu
from jax.experimental.pallas import tpu_sc as plsc

N_NODES = 10000
D_OUT = 128
NC = 2
NS = 16
C = 80
NB = 4
NQ = 8
BM = 2000


def _matmul_body(x_ref, w_ref, o_ref):
    o_ref[...] = jnp.dot(x_ref[...], w_ref[...],
                         preferred_element_type=jnp.float32)


def _matmul(x, w):
    m, k = x.shape
    _, n = w.shape
    return pl.pallas_call(
        _matmul_body,
        grid=(m // BM,),
        in_specs=[
            pl.BlockSpec((BM, k), lambda i: (i, 0)),
            pl.BlockSpec((k, n), lambda i: (0, 0)),
        ],
        out_specs=pl.BlockSpec((BM, n), lambda i: (i, 0)),
        out_shape=jax.ShapeDtypeStruct((m, n), jnp.float32),
    )(x, w)


def _combine2_body(a_ref, b_ref, o_ref):
    o_ref[...] = a_ref[...] + b_ref[...]


def _combine2(parts):
    return pl.pallas_call(
        _combine2_body,
        grid=(N_NODES // BM,),
        in_specs=[
            pl.BlockSpec((BM, D_OUT), lambda i: (i, 0)),
            pl.BlockSpec((BM, D_OUT), lambda i: (i + N_NODES // BM, 0)),
        ],
        out_specs=pl.BlockSpec((BM, D_OUT), lambda i: (i, 0)),
        out_shape=jax.ShapeDtypeStruct((N_NODES, D_OUT), jnp.float32),
    )(parts, parts)


def _final_body(z_ref, a_ref, b_ref, bias_ref, o_ref):
    o_ref[...] = z_ref[...] + a_ref[...] + b_ref[...] + bias_ref[...]


def _final(z1, parts2, bias2d):
    return pl.pallas_call(
        _final_body,
        grid=(N_NODES // BM,),
        in_specs=[
            pl.BlockSpec((BM, D_OUT), lambda i: (i, 0)),
            pl.BlockSpec((BM, D_OUT), lambda i: (i, 0)),
            pl.BlockSpec((BM, D_OUT), lambda i: (i + N_NODES // BM, 0)),
            pl.BlockSpec((1, D_OUT), lambda i: (0, 0)),
        ],
        out_specs=pl.BlockSpec((BM, D_OUT), lambda i: (i, 0)),
        out_shape=jax.ShapeDtypeStruct((N_NODES, D_OUT), jnp.float32),
    )(z1, parts2, parts2, bias2d)


def _make_spmm(n_edges):
    ept = n_edges // (NC * NS)
    rpt = ept // C
    mesh = plsc.VectorSubcoreMesh(core_axis_name="c", subcore_axis_name="s")

    ZR = 40
    scratch = (
        [pltpu.VMEM((C, D_OUT), jnp.float32) for _ in range(NB)]
        + [pltpu.VMEM((C,), jnp.int32) for _ in range(NQ)]
        + [pltpu.VMEM((C,), jnp.int32) for _ in range(NQ)]
        + [pltpu.VMEM((C,), jnp.float32) for _ in range(NQ)]
        + [pltpu.VMEM((ZR, D_OUT), jnp.float32)]
        + [pltpu.VMEM_SHARED((N_NODES, D_OUT), jnp.float32)]
        + [pltpu.SemaphoreType.DMA] * (2 * NB + NQ + 1)
    )

    @functools.partial(
        pl.kernel,
        out_type=jax.ShapeDtypeStruct((NC * N_NODES, D_OUT), jnp.float32),
        mesh=mesh,
        scratch_types=scratch,
    )
    def spmm(table, dst_e, src_e, w_e, out, *scr):
        rows = scr[:NB]
        src_v = scr[NB:NB + NQ]
        dst_v = scr[NB + NQ:NB + 2 * NQ]
        w_v = scr[NB + 2 * NQ:NB + 3 * NQ]
        zbuf = scr[NB + 3 * NQ]
        acc = scr[NB + 3 * NQ + 1]
        sems = scr[NB + 3 * NQ + 2:]
        gsem = sems[:NB]
        ssem = sems[NB:2 * NB]
        isem = sems[2 * NB:2 * NB + NQ]
        zsem = sems[2 * NB + NQ]
        c = lax.axis_index("c")
        s = lax.axis_index("s")
        wid = c * NS + s
        base = wid * ept
        nrow_chunks = N_NODES // C
        nz = N_NODES // ZR

        def start_idx(i, q):
            off = base + i * C
            pltpu.async_copy(src_e.at[pl.ds(off, C)], src_v[q], isem[q])
            pltpu.async_copy(dst_e.at[pl.ds(off, C)], dst_v[q], isem[q])
            pltpu.async_copy(w_e.at[pl.ds(off, C)], w_v[q], isem[q])

        def wait_idx(i, q):
            off = base + i * C
            pltpu.make_async_copy(src_e.at[pl.ds(off, C)], src_v[q],
                                  isem[q]).wait()
            pltpu.make_async_copy(dst_e.at[pl.ds(off, C)], dst_v[q],
                                  isem[q]).wait()
            pltpu.make_async_copy(w_e.at[pl.ds(off, C)], w_v[q],
                                  isem[q]).wait()

        def start_gather(b, q):
            pltpu.async_copy(table.at[src_v[q]], rows[b], gsem[b])

        def wait_gather(b, q):
            pltpu.make_async_copy(table.at[src_v[q]], rows[b],
                                  gsem[b]).wait()

        def start_scatter(b, q):
            pltpu.async_copy(rows[b], acc.at[dst_v[q]], ssem[b], add=True)

        def wait_scatter(b, q):
            pltpu.make_async_copy(rows[b], acc.at[dst_v[q]],
                                  ssem[b]).wait()

        def scale(b, q):
            def group(g, inner):
                wvec = w_v[q][pl.ds(g * 16, 16)]
                for e16 in range(16):
                    e = g * 16 + e16
                    ws = wvec[e16]
                    for j in range(D_OUT // 16):
                        sl = pl.ds(j * 16, 16)
                        rows[b][e, sl] = rows[b][e, sl] * ws
                return inner
            lax.fori_loop(0, C // 16, group, 0)

        for i in range(4):
            start_idx(i, i)
        for i in range(2):
            wait_idx(i, i)
            start_gather(i, i)

        def zfill(e, carry):
            for j in range(D_OUT // 16):
                zbuf[e, pl.ds(j * 16, 16)] = jnp.zeros((16,), jnp.float32)
            return carry
        lax.fori_loop(0, ZR, zfill, 0)

        def zcopy(k, carry):
            idx = s + k * NS

            @pl.when(idx < nz)
            def _():
                pltpu.async_copy(zbuf, acc.at[pl.ds(idx * ZR, ZR)], zsem)
            return carry
        lax.fori_loop(0, pl.cdiv(nz, NS), zcopy, 0)

        def zdrain(k, carry):
            idx = s + k * NS

            @pl.when(idx < nz)
            def _():
                pltpu.make_async_copy(zbuf, acc.at[pl.ds(idx * ZR, ZR)],
                                      zsem).wait()
            return carry
        lax.fori_loop(0, pl.cdiv(nz, NS), zdrain, 0)
        plsc.subcore_barrier()

        def oct_body(k, carry):
            for u in range(8):
                ch = 8 * k + u
                b = u % NB
                wait_gather(b, u)
                scale(b, u)
                start_scatter(b, u)
                wm = (u + 2) % NB
                if u < 2:
                    @pl.when(k > 0)
                    def _(wm=wm, u=u):
                        wait_scatter(wm, (u - 2) % NQ)
                else:
                    wait_scatter(wm, (u - 2) % NQ)
                start_idx(ch + 4, (u + 4) % NQ)
                wait_idx(ch + 2, (u + 2) % NQ)
                start_gather((u + 2) % NB, (u + 2) % NQ)
            return carry
        lax.fori_loop(0, (rpt - 5) // 8, oct_body, 0)

        for ch in range(rpt - 5, rpt):
            u = ch % NQ
            b = u % NB
            wait_gather(b, u)
            scale(b, u)
            start_scatter(b, u)
            if ch + 2 < rpt:
                wait_scatter((u + 2) % NB, (u - 2) % NQ)
                if ch + 4 < rpt:
                    start_idx(ch + 4, (u + 4) % NQ)
                wait_idx(ch + 2, (u + 2) % NQ)
                start_gather((u + 2) % NB, (u + 2) % NQ)
        for ch in range(rpt - 4, rpt):
            u = ch % NQ
            wait_scatter(u % NB, u)
        plsc.subcore_barrier()

        def pubcopy(k, carry):
            idx = s + k * NS

            @pl.when(idx < nrow_chunks)
            def _():
                pltpu.async_copy(acc.at[pl.ds(idx * C, C)],
                                 out.at[pl.ds(c * N_NODES + idx * C, C)],
                                 gsem[0])
            return carry
        lax.fori_loop(0, pl.cdiv(nrow_chunks, NS), pubcopy, 0)

        def pubdrain(k, carry):
            idx = s + k * NS

            @pl.when(idx < nrow_chunks)
            def _():
                pltpu.make_async_copy(
                    acc.at[pl.ds(idx * C, C)],
                    out.at[pl.ds(c * N_NODES + idx * C, C)], gsem[0]).wait()
            return carry
        lax.fori_loop(0, pl.cdiv(nrow_chunks, NS), pubdrain, 0)

    return spmm


def kernel(x, edge_index, edge_weight, W, b):
    dst = edge_index[0]
    src = edge_index[1]
    n_edges = edge_weight.shape[0]

    y = _matmul(x, W)
    spmm = _make_spmm(n_edges)
    parts1 = spmm(y, dst, src, edge_weight)
    z1 = _combine2(parts1)
    parts2 = spmm(z1, dst, src, edge_weight)
    bias2d = jnp.reshape(b, (1, D_OUT))
    return _final(z1, parts2, bias2d)


# speedup vs baseline: 1.0346x; 1.0153x over previous
import functools

import jax
import jax.numpy as jnp
from jax import lax
from jax.experimental import pallas as pl
from jax.experimental.pallas import tpu as pltpu
from jax.experimental.pallas import tpu_sc as plsc

N_NODES = 10000
D_OUT = 128
NC = 2
NS = 16
C = 80
NB = 4
NQ = 8
BM = 5000


def _matmul_body(x_ref, w_ref, o_ref):
    o_ref[...] = jnp.dot(x_ref[...], w_ref[...],
                         preferred_element_type=jnp.float32)


def _matmul(x, w):
    m, k = x.shape
    _, n = w.shape
    return pl.pallas_call(
        _matmul_body,
        grid=(m // BM,),
        in_specs=[
            pl.BlockSpec((BM, k), lambda i: (i, 0)),
            pl.BlockSpec((k, n), lambda i: (0, 0)),
        ],
        out_specs=pl.BlockSpec((BM, n), lambda i: (i, 0)),
        out_shape=jax.ShapeDtypeStruct((m, n), jnp.float32),
    )(x, w)


def _combine2_body(a_ref, b_ref, o_ref):
    o_ref[...] = a_ref[...] + b_ref[...]


def _combine2(parts):
    return pl.pallas_call(
        _combine2_body,
        grid=(N_NODES // BM,),
        in_specs=[
            pl.BlockSpec((BM, D_OUT), lambda i: (i, 0)),
            pl.BlockSpec((BM, D_OUT), lambda i: (i + N_NODES // BM, 0)),
        ],
        out_specs=pl.BlockSpec((BM, D_OUT), lambda i: (i, 0)),
        out_shape=jax.ShapeDtypeStruct((N_NODES, D_OUT), jnp.float32),
    )(parts, parts)


def _final_body(z_ref, a_ref, b_ref, bias_ref, o_ref):
    o_ref[...] = z_ref[...] + a_ref[...] + b_ref[...] + bias_ref[...]


def _final(z1, parts2, bias2d):
    return pl.pallas_call(
        _final_body,
        grid=(N_NODES // BM,),
        in_specs=[
            pl.BlockSpec((BM, D_OUT), lambda i: (i, 0)),
            pl.BlockSpec((BM, D_OUT), lambda i: (i, 0)),
            pl.BlockSpec((BM, D_OUT), lambda i: (i + N_NODES // BM, 0)),
            pl.BlockSpec((1, D_OUT), lambda i: (0, 0)),
        ],
        out_specs=pl.BlockSpec((BM, D_OUT), lambda i: (i, 0)),
        out_shape=jax.ShapeDtypeStruct((N_NODES, D_OUT), jnp.float32),
    )(z1, parts2, parts2, bias2d)


def _make_spmm(n_edges):
    ept = n_edges // (NC * NS)
    rpt = ept // C
    mesh = plsc.VectorSubcoreMesh(core_axis_name="c", subcore_axis_name="s")

    ZR = 40
    scratch = (
        [pltpu.VMEM((C, D_OUT), jnp.float32) for _ in range(NB)]
        + [pltpu.VMEM((C,), jnp.int32) for _ in range(NQ)]
        + [pltpu.VMEM((C,), jnp.int32) for _ in range(NQ)]
        + [pltpu.VMEM((C,), jnp.float32) for _ in range(NQ)]
        + [pltpu.VMEM((ZR, D_OUT), jnp.float32)]
        + [pltpu.VMEM_SHARED((N_NODES, D_OUT), jnp.float32)]
        + [pltpu.SemaphoreType.DMA] * (2 * NB + NQ + 1)
    )

    @functools.partial(
        pl.kernel,
        out_type=jax.ShapeDtypeStruct((NC * N_NODES, D_OUT), jnp.float32),
        mesh=mesh,
        scratch_types=scratch,
    )
    def spmm(table, dst_e, src_e, w_e, out, *scr):
        rows = scr[:NB]
        src_v = scr[NB:NB + NQ]
        dst_v = scr[NB + NQ:NB + 2 * NQ]
        w_v = scr[NB + 2 * NQ:NB + 3 * NQ]
        zbuf = scr[NB + 3 * NQ]
        acc = scr[NB + 3 * NQ + 1]
        sems = scr[NB + 3 * NQ + 2:]
        gsem = sems[:NB]
        ssem = sems[NB:2 * NB]
        isem = sems[2 * NB:2 * NB + NQ]
        zsem = sems[2 * NB + NQ]
        c = lax.axis_index("c")
        s = lax.axis_index("s")
        wid = c * NS + s
        base = wid * ept
        nrow_chunks = N_NODES // C
        nz = N_NODES // ZR

        def start_idx(i, q):
            off = base + i * C
            pltpu.async_copy(src_e.at[pl.ds(off, C)], src_v[q], isem[q])
            pltpu.async_copy(dst_e.at[pl.ds(off, C)], dst_v[q], isem[q])
            pltpu.async_copy(w_e.at[pl.ds(off, C)], w_v[q], isem[q])

        def wait_idx(i, q):
            off = base + i * C
            pltpu.make_async_copy(src_e.at[pl.ds(off, C)], src_v[q],
                                  isem[q]).wait()
            pltpu.make_async_copy(dst_e.at[pl.ds(off, C)], dst_v[q],
                                  isem[q]).wait()
            pltpu.make_async_copy(w_e.at[pl.ds(off, C)], w_v[q],
                                  isem[q]).wait()

        def start_gather(b, q):
            pltpu.async_copy(table.at[src_v[q]], rows[b], gsem[b])

        def wait_gather(b, q):
            pltpu.make_async_copy(table.at[src_v[q]], rows[b],
                                  gsem[b]).wait()

        def start_scatter(b, q):
            pltpu.async_copy(rows[b], acc.at[dst_v[q]], ssem[b], add=True)

        def wait_scatter(b, q):
            pltpu.make_async_copy(rows[b], acc.at[dst_v[q]],
                                  ssem[b]).wait()

        def scale(b, q):
            def group(g, inner):
                wvec = w_v[q][pl.ds(g * 16, 16)]
                for e16 in range(16):
                    e = g * 16 + e16
                    ws = wvec[e16]
                    for j in range(D_OUT // 16):
                        sl = pl.ds(j * 16, 16)
                        rows[b][e, sl] = rows[b][e, sl] * ws
                return inner
            lax.fori_loop(0, C // 16, group, 0)

        for i in range(4):
            start_idx(i, i)
        for i in range(2):
            wait_idx(i, i)
            start_gather(i, i)

        def zfill(e, carry):
            for j in range(D_OUT // 16):
                zbuf[e, pl.ds(j * 16, 16)] = jnp.zeros((16,), jnp.float32)
            return carry
        lax.fori_loop(0, ZR, zfill, 0)

        def zcopy(k, carry):
            idx = s + k * NS

            @pl.when(idx < nz)
            def _():
                pltpu.async_copy(zbuf, acc.at[pl.ds(idx * ZR, ZR)], zsem)
            return carry
        lax.fori_loop(0, pl.cdiv(nz, NS), zcopy, 0)

        def zdrain(k, carry):
            idx = s + k * NS

            @pl.when(idx < nz)
            def _():
                pltpu.make_async_copy(zbuf, acc.at[pl.ds(idx * ZR, ZR)],
                                      zsem).wait()
            return carry
        lax.fori_loop(0, pl.cdiv(nz, NS), zdrain, 0)
        plsc.subcore_barrier()

        def oct_body(k, carry):
            for u in range(8):
                ch = 8 * k + u
                b = u % NB
                wait_gather(b, u)
                scale(b, u)
                start_scatter(b, u)
                wm = (u + 2) % NB
                if u < 2:
                    @pl.when(k > 0)
                    def _(wm=wm, u=u):
                        wait_scatter(wm, (u - 2) % NQ)
                else:
                    wait_scatter(wm, (u - 2) % NQ)
                start_idx(ch + 4, (u + 4) % NQ)
                wait_idx(ch + 2, (u + 2) % NQ)
                start_gather((u + 2) % NB, (u + 2) % NQ)
            return carry
        lax.fori_loop(0, (rpt - 5) // 8, oct_body, 0)

        for ch in range(rpt - 5, rpt):
            u = ch % NQ
            b = u % NB
            wait_gather(b, u)
            scale(b, u)
            start_scatter(b, u)
            if ch + 2 < rpt:
                wait_scatter((u + 2) % NB, (u - 2) % NQ)
                if ch + 4 < rpt:
                    start_idx(ch + 4, (u + 4) % NQ)
                wait_idx(ch + 2, (u + 2) % NQ)
                start_gather((u + 2) % NB, (u + 2) % NQ)
        for ch in range(rpt - 4, rpt):
            u = ch % NQ
            wait_scatter(u % NB, u)
        plsc.subcore_barrier()

        def pubcopy(k, carry):
            idx = s + k * NS

            @pl.when(idx < nrow_chunks)
            def _():
                pltpu.async_copy(acc.at[pl.ds(idx * C, C)],
                                 out.at[pl.ds(c * N_NODES + idx * C, C)],
                                 gsem[0])
            return carry
        lax.fori_loop(0, pl.cdiv(nrow_chunks, NS), pubcopy, 0)

        def pubdrain(k, carry):
            idx = s + k * NS

            @pl.when(idx < nrow_chunks)
            def _():
                pltpu.make_async_copy(
                    acc.at[pl.ds(idx * C, C)],
                    out.at[pl.ds(c * N_NODES + idx * C, C)], gsem[0]).wait()
            return carry
        lax.fori_loop(0, pl.cdiv(nrow_chunks, NS), pubdrain, 0)

    return spmm


def kernel(x, edge_index, edge_weight, W, b):
    dst = edge_index[0]
    src = edge_index[1]
    n_edges = edge_weight.shape[0]

    y = _matmul(x, W)
    spmm = _make_spmm(n_edges)
    parts1 = spmm(y, dst, src, edge_weight)
    z1 = _combine2(parts1)
    parts2 = spmm(z1, dst, src, edge_weight)
    bias2d = jnp.reshape(b, (1, D_OUT))
    return _final(z1, parts2, bias2d)
